# baseline (speedup 1.0000x reference)
import functools

import jax
import jax.numpy as jnp
from jax import lax
from jax.experimental import pallas as pl
from jax.experimental.pallas import tpu as pltpu
from jax.experimental.pallas import tpu_sc as plsc

EMB = 64
SCALE = 8.0
LANES = 16

NW = 32
IDX_MINOR = 128
KROW = 4
CHUNK = KROW * IDX_MINOR


def _emb_body(tok_hbm, table_hbm, out_hbm, idx_v, rows_v, gsem):
    n_chunks = tok_hbm.shape[0] // (NW * KROW)
    wid = lax.axis_index("s") * 2 + lax.axis_index("c")
    tok_row_base = wid * (n_chunks * KROW)
    out_base = tok_row_base * IDX_MINOR

    def chunk_body(ci, carry):
        pltpu.sync_copy(tok_hbm.at[pl.ds(tok_row_base + ci * KROW, KROW)], idx_v)
        copies = []
        for j in range(KROW):
            copies.append(
                pltpu.async_copy(
                    table_hbm.at[idx_v.at[j]],
                    rows_v.at[pl.ds(j * IDX_MINOR, IDX_MINOR)],
                    gsem,
                )
            )
        for c in copies:
            c.wait()

        def mul_body(r, carry2):
            for t in range(EMB // LANES):
                sl = (r, pl.ds(t * LANES, LANES))
                rows_v[sl] = rows_v[sl] * SCALE
            return carry2

        lax.fori_loop(0, CHUNK, mul_body, 0, unroll=2)

        pltpu.sync_copy(rows_v, out_hbm.at[pl.ds(out_base + ci * CHUNK, CHUNK)])
        return carry

    lax.fori_loop(0, n_chunks, chunk_body, 0)


def kernel(tokens, table):
    batch, hist = tokens.shape
    n_rows = batch * hist
    tok2d = jnp.reshape(tokens.astype(jnp.int32), (n_rows // IDX_MINOR, IDX_MINOR))

    mesh = plsc.VectorSubcoreMesh(core_axis_name="c", subcore_axis_name="s")
    run = functools.partial(
        pl.kernel,
        mesh=mesh,
        compiler_params=pltpu.CompilerParams(use_tc_tiling_on_sc=False),
        out_type=jax.ShapeDtypeStruct((n_rows, EMB), jnp.float32),
        scratch_types=[
            pltpu.VMEM((KROW, IDX_MINOR), jnp.int32),
            pltpu.VMEM((CHUNK, EMB), jnp.float32),
            pltpu.SemaphoreType.DMA,
        ],
    )(_emb_body)
    out = run(tok2d, table)
    return jnp.reshape(out, (batch, hist, EMB))


# speedup vs baseline: 1.1468x; 1.1468x over previous
import functools

import jax
import jax.numpy as jnp
from jax import lax
from jax.experimental import pallas as pl
from jax.experimental.pallas import tpu as pltpu
from jax.experimental.pallas import tpu_sc as plsc

EMB = 64
SCALE = 8.0
LANES = 16

NW = 32
IDX_MINOR = 128
KROW = 4
CHUNK = KROW * IDX_MINOR
NBUF = 2


def _fire_gathers(table_hbm, idx_v, rows_v, gsem):
    for j in range(KROW):
        pltpu.async_copy(
            table_hbm.at[idx_v.at[j]],
            rows_v.at[pl.ds(j * IDX_MINOR, IDX_MINOR)],
            gsem,
        )


def _drain_gathers(table_hbm, idx_v, rows_v, gsem):
    for j in range(KROW):
        pltpu.make_async_copy(
            table_hbm.at[idx_v.at[j]],
            rows_v.at[pl.ds(j * IDX_MINOR, IDX_MINOR)],
            gsem,
        ).wait()


def _emb_body(tok_hbm, table_hbm, out_hbm,
              idx0, idx1, rows0, rows1, isem0, isem1, gsem0, gsem1,
              ssem0, ssem1):
    idx = (idx0, idx1)
    rows = (rows0, rows1)
    isem = (isem0, isem1)
    gsem = (gsem0, gsem1)
    ssem = (ssem0, ssem1)

    n_chunks = tok_hbm.shape[0] // (NW * KROW)
    wid = lax.axis_index("s") * 2 + lax.axis_index("c")
    tok_row_base = wid * (n_chunks * KROW)
    out_base = tok_row_base * IDX_MINOR

    def idx_copy(ci, b):
        return pltpu.make_async_copy(
            tok_hbm.at[pl.ds(tok_row_base + ci * KROW, KROW)], idx[b], isem[b])

    def out_copy(ci, b):
        return pltpu.make_async_copy(
            rows[b], out_hbm.at[pl.ds(out_base + ci * CHUNK, CHUNK)], ssem[b])

    c0 = idx_copy(0, 0)
    c0.start()
    c0.wait()
    idx_copy(1, 1).start()
    _fire_gathers(table_hbm, idx[0], rows[0], gsem[0])

    def outer(oi, carry):
        for b in range(NBUF):
            ci = oi * NBUF + b
            nb = b ^ 1
            _drain_gathers(table_hbm, idx[b], rows[b], gsem[b])

            @pl.when(ci > 0)
            def _():
                out_copy(0, nb).wait()

            @pl.when(ci + 1 < n_chunks)
            def _():
                idx_copy(0, nb).wait()
                _fire_gathers(table_hbm, idx[nb], rows[nb], gsem[nb])

            @pl.when(ci + 2 < n_chunks)
            def _():
                idx_copy(ci + 2, b).start()

            def mul_body(r, carry2):
                for t in range(EMB // LANES):
                    sl = (r, pl.ds(t * LANES, LANES))
                    rows[b][sl] = rows[b][sl] * SCALE
                return carry2

            lax.fori_loop(0, CHUNK, mul_body, 0, unroll=2)

            out_copy(ci, b).start()
        return carry

    lax.fori_loop(0, n_chunks // NBUF, outer, 0)
    out_copy(0, (n_chunks - 1) % NBUF).wait()


def kernel(tokens, table):
    batch, hist = tokens.shape
    n_rows = batch * hist
    tok2d = jnp.reshape(tokens.astype(jnp.int32), (n_rows // IDX_MINOR, IDX_MINOR))

    mesh = plsc.VectorSubcoreMesh(core_axis_name="c", subcore_axis_name="s")
    run = functools.partial(
        pl.kernel,
        mesh=mesh,
        compiler_params=pltpu.CompilerParams(use_tc_tiling_on_sc=False),
        out_type=jax.ShapeDtypeStruct((n_rows, EMB), jnp.float32),
        scratch_types=[
            pltpu.VMEM((KROW, IDX_MINOR), jnp.int32),
            pltpu.VMEM((KROW, IDX_MINOR), jnp.int32),
            pltpu.VMEM((CHUNK, EMB), jnp.float32),
            pltpu.VMEM((CHUNK, EMB), jnp.float32),
            pltpu.SemaphoreType.DMA,
            pltpu.SemaphoreType.DMA,
            pltpu.SemaphoreType.DMA,
            pltpu.SemaphoreType.DMA,
            pltpu.SemaphoreType.DMA,
            pltpu.SemaphoreType.DMA,
        ],
    )(_emb_body)
    out = run(tok2d, table)
    return jnp.reshape(out, (batch, hist, EMB))
